# baseline (speedup 1.0000x reference)
import functools

import jax
import jax.numpy as jnp
from jax import lax
from jax.experimental import pallas as pl
from jax.experimental.pallas import tpu as pltpu
from jax.experimental.pallas import tpu_sc as plsc

_L = 16


@functools.lru_cache(maxsize=None)
def _build(R, V, D, S):
    info = plsc.get_sparse_core_info()
    nc, ns = info.num_cores, info.num_subcores
    nw = nc * ns
    chunk = 4 * S
    rows_w = R // nw
    nch = rows_w // chunk
    assert rows_w % chunk == 0 and R % nw == 0 and D % _L == 0

    subs = []
    off = 0
    while off < chunk:
        sz = min(128, chunk - off)
        subs.append((off, sz))
        off += sz
    nsub = len(subs)

    mesh = plsc.VectorSubcoreMesh(core_axis_name="c", subcore_axis_name="s")

    @functools.partial(
        pl.kernel,
        mesh=mesh,
        compiler_params=pltpu.CompilerParams(use_tc_tiling_on_sc=False),
        out_type=jax.ShapeDtypeStruct((R, D), jnp.float32),
        scratch_types=[
            pltpu.VMEM((2, nsub, 128), jnp.int32),
            pltpu.VMEM((2, chunk + 8, D), jnp.float32),
            pltpu.VMEM((D,), jnp.float32),
            pltpu.SemaphoreType.DMA,
            pltpu.SemaphoreType.DMA,
            pltpu.SemaphoreType.DMA,
            pltpu.SemaphoreType.DMA,
        ],
    )
    def gather_kernel(tab_hbm, idx_hbm, start_hbm, out_hbm,
                      idx_v, rows_v, start_v, sg0, sg1, so0, so1):
        wid = lax.axis_index("s") * nc + lax.axis_index("c")
        base = pl.multiple_of(wid * rows_w, chunk)
        pltpu.sync_copy(start_hbm, start_v)
        st = [start_v[pl.ds(i * _L, _L)] for i in range(D // _L)]
        sems_g = [sg0, sg1]
        sems_o = [so0, so1]
        out_h = [None, None]
        for c in range(nch):
            b = c % 2
            g = pl.multiple_of(base + c * chunk, chunk)
            if out_h[b] is not None:
                out_h[b].wait()
                out_h[b] = None
            for j, (o, sz) in enumerate(subs):
                pltpu.sync_copy(idx_hbm.at[pl.ds(g + o, sz)],
                                idx_v.at[b, j, pl.ds(0, sz)])
            handles = []
            for j, (o, sz) in enumerate(subs):
                handles.append(pltpu.async_copy(
                    tab_hbm.at[idx_v.at[b, j, pl.ds(0, sz)]],
                    rows_v.at[b, pl.ds(o + 1, sz)],
                    sems_g[b]))
            for h in handles:
                h.wait()
            for r in range(0, chunk, S):
                for i in range(D // _L):
                    rows_v[b, r, pl.ds(i * _L, _L)] = st[i]
            out_h[b] = pltpu.async_copy(
                rows_v.at[b, pl.ds(0, chunk)], out_hbm.at[pl.ds(g, chunk)],
                sems_o[b])
        for b in range(2):
            if out_h[b] is not None:
                out_h[b].wait()

    return gather_kernel


def kernel(img, formula, E, start_token):
    B, S = formula.shape
    V, D = E.shape
    fm = formula.reshape(-1).astype(jnp.int32)
    out = _build(B * S, V, D, S)(E, fm, start_token)
    return out.reshape(B, S, D)


# speedup vs baseline: 1.0123x; 1.0123x over previous
import functools

import jax
import jax.numpy as jnp
from jax import lax
from jax.experimental import pallas as pl
from jax.experimental.pallas import tpu as pltpu
from jax.experimental.pallas import tpu_sc as plsc

_L = 16


@functools.lru_cache(maxsize=None)
def _build(R, V, D, S):
    info = plsc.get_sparse_core_info()
    nc, ns = info.num_cores, info.num_subcores
    nw = nc * ns
    chunk = 4 * S
    rows_w = R // nw
    nch = rows_w // chunk
    assert rows_w % chunk == 0 and R % nw == 0 and D % _L == 0

    rows_per_chunk = chunk // S
    subs = []
    for j in range(rows_per_chunk):
        off = 0
        while off < S:
            sz = min(128, S - off)
            subs.append((j, off, sz))
            off += sz
    nsub = len(subs)

    mesh = plsc.VectorSubcoreMesh(core_axis_name="c", subcore_axis_name="s")

    @functools.partial(
        pl.kernel,
        mesh=mesh,
        compiler_params=pltpu.CompilerParams(use_tc_tiling_on_sc=False),
        out_type=jax.ShapeDtypeStruct((R, D), jnp.float32),
        scratch_types=[
            pltpu.VMEM((2, rows_per_chunk, S), jnp.int32),
            pltpu.VMEM((2, chunk + 8, D), jnp.float32),
            pltpu.VMEM((D,), jnp.float32),
            pltpu.SemaphoreType.DMA,
            pltpu.SemaphoreType.DMA,
            pltpu.SemaphoreType.DMA,
            pltpu.SemaphoreType.DMA,
        ],
    )
    def gather_kernel(tab_hbm, idx_hbm, start_hbm, out_hbm,
                      idx_v, rows_v, start_v, sg0, sg1, so0, so1):
        wid = lax.axis_index("s") * nc + lax.axis_index("c")
        base = pl.multiple_of(wid * rows_w, chunk)
        fbase = wid * (rows_w // S)
        pltpu.sync_copy(start_hbm, start_v)
        st = [start_v[pl.ds(i * _L, _L)] for i in range(D // _L)]
        sems_g = [sg0, sg1]
        sems_o = [so0, so1]
        out_h = [None, None]
        for c in range(nch):
            b = c % 2
            g = pl.multiple_of(base + c * chunk, chunk)
            if out_h[b] is not None:
                out_h[b].wait()
                out_h[b] = None
            for j in range(rows_per_chunk):
                pltpu.sync_copy(idx_hbm.at[fbase + c * rows_per_chunk + j],
                                idx_v.at[b, j])
            handles = []
            for (j, o, sz) in subs:
                handles.append(pltpu.async_copy(
                    tab_hbm.at[idx_v.at[b, j, pl.ds(o, sz)]],
                    rows_v.at[b, pl.ds(j * S + o + 1, sz)],
                    sems_g[b]))
            for h in handles:
                h.wait()
            for r in range(0, chunk, S):
                for i in range(D // _L):
                    rows_v[b, r, pl.ds(i * _L, _L)] = st[i]
            out_h[b] = pltpu.async_copy(
                rows_v.at[b, pl.ds(0, chunk)], out_hbm.at[pl.ds(g, chunk)],
                sems_o[b])
        for b in range(2):
            if out_h[b] is not None:
                out_h[b].wait()

    return gather_kernel


def kernel(img, formula, E, start_token):
    B, S = formula.shape
    V, D = E.shape
    fm = formula.astype(jnp.int32)
    out = _build(B * S, V, D, S)(E, fm, start_token)
    return out.reshape(B, S, D)


# speedup vs baseline: 1.1006x; 1.0872x over previous
import functools

import jax
import jax.numpy as jnp
from jax import lax
from jax.experimental import pallas as pl
from jax.experimental.pallas import tpu as pltpu
from jax.experimental.pallas import tpu_sc as plsc

_L = 16


@functools.lru_cache(maxsize=None)
def _build(R, V, D, S):
    info = plsc.get_sparse_core_info()
    nc, ns = info.num_cores, info.num_subcores
    nw = nc * ns
    chunk = 4 * S
    rows_w = R // nw
    nch = rows_w // chunk
    assert rows_w % chunk == 0 and R % nw == 0 and D % _L == 0

    rows_per_chunk = chunk // S
    subs = []
    for j in range(rows_per_chunk):
        off = 0
        while off < S:
            sz = min(128, S - off)
            subs.append((j, off, sz))
            off += sz
    nsub = len(subs)

    mesh = plsc.VectorSubcoreMesh(core_axis_name="c", subcore_axis_name="s")

    @functools.partial(
        pl.kernel,
        mesh=mesh,
        compiler_params=pltpu.CompilerParams(use_tc_tiling_on_sc=False),
        out_type=jax.ShapeDtypeStruct((R, D), jnp.float32),
        scratch_types=[
            pltpu.VMEM((2, rows_per_chunk, S), jnp.int32),
            pltpu.VMEM((2, chunk + 8, D), jnp.float32),
            pltpu.VMEM((D,), jnp.float32),
            pltpu.SemaphoreType.DMA,
            pltpu.SemaphoreType.DMA,
            pltpu.SemaphoreType.DMA,
            pltpu.SemaphoreType.DMA,
        ],
    )
    def gather_kernel(tab_hbm, idx_hbm, start_hbm, out_hbm,
                      idx_v, rows_v, start_v, sg0, sg1, so0, so1):
        wid = lax.axis_index("s") * nc + lax.axis_index("c")
        base = pl.multiple_of(wid * rows_w, chunk)
        fbase = wid * (rows_w // S)
        pltpu.sync_copy(start_hbm, start_v)
        st = [start_v[pl.ds(i * _L, _L)] for i in range(D // _L)]
        sems_g = [sg0, sg1]
        sems_o = [so0, so1]
        out_h = [None, None]
        for c in range(nch):
            b = c % 2
            g = pl.multiple_of(base + c * chunk, chunk)
            if out_h[b] is not None:
                out_h[b].wait()
                out_h[b] = None
            for j in range(rows_per_chunk):
                pltpu.sync_copy(idx_hbm.at[fbase + c * rows_per_chunk + j],
                                idx_v.at[b, j])
            handles = []
            for (j, o, sz) in subs:
                handles.append(pltpu.async_copy(
                    tab_hbm.at[idx_v.at[b, j, pl.ds(o, sz)]],
                    rows_v.at[b, pl.ds(j * S + o + 1, sz)],
                    sems_g[b]))
            for h in handles:
                h.wait()
            for r in range(0, chunk, S):
                for i in range(D // _L):
                    rows_v[b, r, pl.ds(i * _L, _L)] = st[i]
            out_h[b] = pltpu.async_copy(
                rows_v.at[b, pl.ds(0, chunk)], out_hbm.at[pl.ds(g, chunk)],
                sems_o[b])
        for b in range(2):
            if out_h[b] is not None:
                out_h[b].wait()

    return gather_kernel


def kernel(img, formula, E, start_token):
    B, S = formula.shape
    V, D = E.shape
    tab = jnp.pad(E, ((0, 0), (0, D))).reshape(2 * V, D)
    fm = formula.astype(jnp.int32) * 2
    out = _build(B * S, 2 * V, D, S)(tab, fm, start_token)
    return out.reshape(B, S, D)
